# baseline (speedup 1.0000x reference)
import functools

import jax
import jax.numpy as jnp
from jax import lax
from jax.experimental import pallas as pl
from jax.experimental.pallas import tpu as pltpu
from jax.experimental.pallas import tpu_sc as plsc

DIM = 64
GAMMA = 12.0
BLK = 128
LANES = 16
NC = 2
NS = 16
NW = NC * NS

N_ENT = 1000000
N_COLS = N_ENT // 128
N_TAIL = N_ENT - N_COLS * 128
N_REL = 1000
N_CLAUSES = 16384
N_SINGLES = 8192
N_CONJ = 8192

N_DCOLS = N_COLS // 2
DCOLS_BASE = N_DCOLS // NW
DCOLS_EXTRA = N_DCOLS % NW
DC_ENT = 256


def _transpose_kernel(ent_t, tailf, ent_flat,
                      ckb0, ckb1, obuf0, obuf1, tbuf, sem, osem):
    wid = lax.axis_index("s") * NC + lax.axis_index("c")
    lane = lax.broadcasted_iota(jnp.int32, (LANES,), 0)

    c0 = wid * DCOLS_BASE

    @pl.when(wid == 0)
    def _():
        pltpu.sync_copy(tailf, tbuf)
        pltpu.sync_copy(tbuf, ent_flat.at[pl.ds(N_COLS * 128 * DIM,
                                                N_TAIL * DIM)])

    ckbs = (ckb0, ckb1)
    obufs = (obuf0, obuf1)
    rots = [(lane + k) & 15 for k in range(LANES)]
    rowvs = [lane + LANES * db for db in range(DIM // LANES)]

    def issue_col(dc_idx, slot):
        col = pl.multiple_of(dc_idx * DC_ENT, DC_ENT)
        for a in range(8):
            pltpu.async_copy(ent_t.at[pl.ds(8 * a, 8), pl.ds(col, DC_ENT)],
                             ckbs[slot].at[pl.ds(8 * a, 8), :], sem)

    def drain_col(slot):
        pltpu.make_async_copy(ent_t.at[pl.ds(0, DIM), pl.ds(0, DC_ENT)],
                              ckbs[slot], sem).wait()

    def transpose_col(slot):
        ckb = ckbs[slot]
        ob = obufs[slot]

        @plsc.parallel_loop(0, DC_ENT // LANES, unroll=8)
        def _(e0i):
            e0 = e0i * LANES
            for k in range(LANES):
                colv = e0 + rots[k]
                sbase = colv << 6
                for db in range(DIM // LANES):
                    v = plsc.load_gather(ckb, [rowvs[db], colv])
                    plsc.store_scatter(ob, [sbase + rowvs[db]], v)

    def flush_col(dc_idx, slot):
        off = pl.multiple_of(dc_idx * (DC_ENT * DIM), DC_ENT * DIM)
        pltpu.async_copy(obufs[slot],
                         ent_flat.at[pl.ds(off, DC_ENT * DIM)], osem)

    def drain_flush(slot):
        pltpu.make_async_copy(ent_flat.at[pl.ds(0, DC_ENT * DIM)],
                              obufs[slot], osem).wait()

    issue_col(c0, 0)
    issue_col(c0 + 1, 1)

    def pair_body(p, _):
        for s in range(2):
            ci = p * 2 + s
            drain_col(s)

            @pl.when(ci >= 2)
            def _():
                drain_flush(s)

            transpose_col(s)
            flush_col(c0 + ci, s)

            @pl.when(ci + 2 < DCOLS_BASE)
            def _():
                issue_col(c0 + ci + 2, s)
        return 0

    lax.fori_loop(0, DCOLS_BASE // 2, pair_body, 0)
    drain_flush(0)
    drain_flush(1)

    @pl.when(wid < DCOLS_EXTRA)
    def _():
        ecol = NW * DCOLS_BASE + wid
        issue_col(ecol, 0)
        drain_col(0)
        transpose_col(0)
        off = pl.multiple_of(ecol * (DC_ENT * DIM), DC_ENT * DIM)
        pltpu.sync_copy(obuf0, ent_flat.at[pl.ds(off, DC_ENT * DIM)])


def _scores_kernel(ent, rel, ch, ct, cr, ph, pt, pr,
                   c1h, c1t, c1r, c2h, c2t, c2r,
                   out_c, out_p, out_j,
                   hidx0, tidx0, ridx0, hrows0, trows0, rrows0,
                   hidx1, tidx1, ridx1, hrows1, trows1, rrows1,
                   sbuf, sbuf2, sem):
    wid = lax.axis_index("s") * NC + lax.axis_index("c")
    lane = lax.broadcasted_iota(jnp.int32, (LANES,), 0)
    idxs = ((hidx0, tidx0, ridx0), (hidx1, tidx1, ridx1))
    rows = ((hrows0, trows0, rrows0), (hrows1, trows1, rrows1))

    jobs = []
    npw = N_CLAUSES // NW
    for b in range(npw // BLK):
        jobs.append((ch, ct, cr, npw, b, sbuf, b * BLK, out_c))
    npw = N_SINGLES // NW
    for b in range(npw // BLK):
        jobs.append((ph, pt, pr, npw, b, sbuf, b * BLK, out_p))
    npw = N_CONJ // NW
    for b in range(npw // BLK):
        jobs.append((c1h, c1t, c1r, npw, b, sbuf, b * BLK, None))
    for b in range(npw // BLK):
        jobs.append((c2h, c2t, c2r, npw, b, sbuf2, b * BLK, None))

    def issue(job, slot):
        h_hbm, t_hbm, r_hbm, npw, b, _, _, _ = job
        base = wid * npw + b * BLK
        hi, ti, ri = idxs[slot]
        hr, tr, rr = rows[slot]
        pltpu.sync_copy(h_hbm.at[pl.ds(base, BLK)], hi)
        pltpu.sync_copy(t_hbm.at[pl.ds(base, BLK)], ti)
        pltpu.sync_copy(r_hbm.at[pl.ds(base, BLK)], ri)
        pltpu.async_copy(ent.at[hi], hr, sem)
        pltpu.async_copy(ent.at[ti], tr, sem)
        pltpu.async_copy(rel.at[ri], rr, sem)

    def drain(slot):
        hr, tr, rr = rows[slot]
        pltpu.make_async_copy(ent.at[pl.ds(0, BLK)], hr, sem).wait()
        pltpu.make_async_copy(ent.at[pl.ds(0, BLK)], tr, sem).wait()
        pltpu.make_async_copy(rel.at[pl.ds(0, BLK)], rr, sem).wait()

    def compute(job, slot):
        hr, tr, rr = rows[slot]
        out_buf, s_off = job[5], job[6]
        for g in range(BLK // LANES):

            def a_body(j, svec):
                a = g * LANES + j
                acc = jnp.zeros((LANES,), jnp.float32)
                for k in range(DIM // LANES):
                    sl = pl.ds(k * LANES, LANES)
                    acc = acc + jnp.abs(hr[a, sl] + rr[a, sl] - tr[a, sl])
                s = GAMMA - jnp.sum(acc)
                return jnp.where(lane == j, s, svec)

            svec = lax.fori_loop(0, LANES, a_body,
                                 jnp.zeros((LANES,), jnp.float32))
            out_buf[pl.ds(s_off + g * LANES, LANES)] = svec

    issue(jobs[0], 0)
    for i, job in enumerate(jobs):
        slot = i % 2
        if i + 1 < len(jobs):
            issue(jobs[i + 1], 1 - slot)
        drain(slot)
        compute(job, slot)
        h_hbm, t_hbm, r_hbm, npw, b, _, s_off, out_hbm = job
        if out_hbm is not None:
            base = wid * npw + b * BLK
            pltpu.sync_copy(sbuf.at[pl.ds(s_off, BLK)],
                            out_hbm.at[pl.ds(base, BLK)])

    npw = N_CONJ // NW
    for v in range(npw // LANES):
        sl = pl.ds(v * LANES, LANES)
        sbuf[sl] = jnp.minimum(sbuf[sl], sbuf2[sl])
    for b in range(npw // BLK):
        base = wid * npw + b * BLK
        pltpu.sync_copy(sbuf.at[pl.ds(b * BLK, BLK)],
                        out_j.at[pl.ds(base, BLK)])


@functools.cache
def _build_transpose():
    mesh = plsc.VectorSubcoreMesh(core_axis_name="c", subcore_axis_name="s")
    return pl.kernel(
        _transpose_kernel,
        mesh=mesh,
        compiler_params=pltpu.CompilerParams(
            needs_layout_passes=False, use_tc_tiling_on_sc=True),
        out_type=jax.ShapeDtypeStruct((N_ENT * DIM,), jnp.float32),
        scratch_types=[
            pltpu.VMEM((DIM, DC_ENT), jnp.float32),
            pltpu.VMEM((DIM, DC_ENT), jnp.float32),
            pltpu.VMEM((DC_ENT * DIM,), jnp.float32),
            pltpu.VMEM((DC_ENT * DIM,), jnp.float32),
            pltpu.VMEM((N_TAIL * DIM,), jnp.float32),
            pltpu.SemaphoreType.DMA,
            pltpu.SemaphoreType.DMA,
        ],
    )


@functools.cache
def _build_scores():
    mesh = plsc.VectorSubcoreMesh(core_axis_name="c", subcore_axis_name="s")
    return pl.kernel(
        _scores_kernel,
        mesh=mesh,
        compiler_params=pltpu.CompilerParams(
            needs_layout_passes=False, use_tc_tiling_on_sc=False),
        out_type=[
            jax.ShapeDtypeStruct((N_CLAUSES,), jnp.float32),
            jax.ShapeDtypeStruct((N_SINGLES,), jnp.float32),
            jax.ShapeDtypeStruct((N_CONJ,), jnp.float32),
        ],
        scratch_types=[
            pltpu.VMEM((BLK,), jnp.int32),
            pltpu.VMEM((BLK,), jnp.int32),
            pltpu.VMEM((BLK,), jnp.int32),
            pltpu.VMEM((BLK, DIM), jnp.float32),
            pltpu.VMEM((BLK, DIM), jnp.float32),
            pltpu.VMEM((BLK, DIM), jnp.float32),
            pltpu.VMEM((BLK,), jnp.int32),
            pltpu.VMEM((BLK,), jnp.int32),
            pltpu.VMEM((BLK,), jnp.int32),
            pltpu.VMEM((BLK, DIM), jnp.float32),
            pltpu.VMEM((BLK, DIM), jnp.float32),
            pltpu.VMEM((BLK, DIM), jnp.float32),
            pltpu.VMEM((N_CLAUSES // NW,), jnp.float32),
            pltpu.VMEM((N_CLAUSES // NW,), jnp.float32),
            pltpu.SemaphoreType.DMA,
        ],
    )


def kernel(clause_entity_embedding, relation_embedding,
           concl_heads, concl_tails, concl_rel,
           premise_heads, premise_tails, premise_rel,
           conj_premise_heads1, conj_premise_tails1, conj_premise_rel1,
           conj_premise_heads2, conj_premise_tails2, conj_premise_rel2):
    ent_t = clause_entity_embedding.T
    tail_flat = lax.slice(clause_entity_embedding,
                          (N_COLS * 128, 0), (N_ENT, DIM)).reshape(-1)
    ent_flat = _build_transpose()(ent_t, tail_flat)
    ent_rm = ent_flat.reshape(N_ENT, DIM)
    concl, prem, conj = _build_scores()(
        ent_rm, relation_embedding,
        concl_heads, concl_tails, concl_rel,
        premise_heads, premise_tails, premise_rel,
        conj_premise_heads1, conj_premise_tails1, conj_premise_rel1,
        conj_premise_heads2, conj_premise_tails2, conj_premise_rel2)
    return (concl, prem, conj)


# speedup vs baseline: 1.2602x; 1.2602x over previous
import functools

import jax
import jax.numpy as jnp
from jax import lax
from jax.experimental import pallas as pl
from jax.experimental.pallas import tpu as pltpu
from jax.experimental.pallas import tpu_sc as plsc

DIM = 64
GAMMA = 12.0
BLK = 128
LANES = 16
NC = 2
NS = 16
NW = NC * NS

N_ENT = 1000000
N_COLS = N_ENT // 128
N_TAIL = N_ENT - N_COLS * 128
N_REL = 1000
N_CLAUSES = 16384
N_SINGLES = 8192
N_CONJ = 8192

N_DCOLS = N_COLS // 2
DCOLS_BASE = N_DCOLS // NW
DCOLS_EXTRA = N_DCOLS % NW
DC_ENT = 256


def _transpose_kernel(ent_t, tailf, ent_flat,
                      ckb0, ckb1, obuf0, obuf1, tbuf, sem, osem):
    wid = lax.axis_index("s") * NC + lax.axis_index("c")
    lane = lax.broadcasted_iota(jnp.int32, (LANES,), 0)

    c0 = wid * DCOLS_BASE

    @pl.when(wid == 0)
    def _():
        pltpu.sync_copy(tailf, tbuf)
        pltpu.sync_copy(tbuf, ent_flat.at[pl.ds(N_COLS * 128 * DIM,
                                                N_TAIL * DIM)])

    ckbs = (ckb0, ckb1)
    obufs = (obuf0, obuf1)
    rots = [(lane + k) & 15 for k in range(LANES)]
    rowvs = [lane + LANES * db for db in range(DIM // LANES)]

    def issue_col(dc_idx, slot):
        col = pl.multiple_of(dc_idx * DC_ENT, DC_ENT)
        for a in range(8):
            pltpu.async_copy(ent_t.at[pl.ds(8 * a, 8), pl.ds(col, DC_ENT)],
                             ckbs[slot].at[pl.ds(8 * a, 8), :], sem)

    def drain_col(slot):
        pltpu.make_async_copy(ent_t.at[pl.ds(0, DIM), pl.ds(0, DC_ENT)],
                              ckbs[slot], sem).wait()

    def transpose_col(slot):
        ckb = ckbs[slot]
        ob = obufs[slot]

        @plsc.parallel_loop(0, DC_ENT // LANES, unroll=4)
        def _(e0i):
            e0 = e0i * LANES
            for k in range(LANES):
                colv = e0 + rots[k]
                sbase = colv << 6
                for db in range(DIM // LANES):
                    v = plsc.load_gather(ckb, [rowvs[db], colv])
                    plsc.store_scatter(ob, [sbase + rowvs[db]], v)

    def flush_col(dc_idx, slot):
        off = pl.multiple_of(dc_idx * (DC_ENT * DIM), DC_ENT * DIM)
        pltpu.async_copy(obufs[slot],
                         ent_flat.at[pl.ds(off, DC_ENT * DIM)], osem)

    def drain_flush(slot):
        pltpu.make_async_copy(ent_flat.at[pl.ds(0, DC_ENT * DIM)],
                              obufs[slot], osem).wait()

    issue_col(c0, 0)
    issue_col(c0 + 1, 1)

    def pair_body(p, _):
        for s in range(2):
            ci = p * 2 + s
            drain_col(s)

            @pl.when(ci >= 2)
            def _():
                drain_flush(s)

            transpose_col(s)
            flush_col(c0 + ci, s)

            @pl.when(ci + 2 < DCOLS_BASE)
            def _():
                issue_col(c0 + ci + 2, s)
        return 0

    lax.fori_loop(0, DCOLS_BASE // 2, pair_body, 0)
    drain_flush(0)
    drain_flush(1)

    @pl.when(wid < DCOLS_EXTRA)
    def _():
        ecol = NW * DCOLS_BASE + wid
        issue_col(ecol, 0)
        drain_col(0)
        transpose_col(0)
        off = pl.multiple_of(ecol * (DC_ENT * DIM), DC_ENT * DIM)
        pltpu.sync_copy(obuf0, ent_flat.at[pl.ds(off, DC_ENT * DIM)])


def _scores_kernel(ent, rel, ch, ct, cr, ph, pt, pr,
                   c1h, c1t, c1r, c2h, c2t, c2r,
                   out_c, out_p, out_j,
                   hidx0, tidx0, ridx0, hrows0, trows0, rrows0,
                   hidx1, tidx1, ridx1, hrows1, trows1, rrows1,
                   sbuf, sbuf2, sem):
    wid = lax.axis_index("s") * NC + lax.axis_index("c")
    lane = lax.broadcasted_iota(jnp.int32, (LANES,), 0)
    idxs = ((hidx0, tidx0, ridx0), (hidx1, tidx1, ridx1))
    rows = ((hrows0, trows0, rrows0), (hrows1, trows1, rrows1))

    jobs = []
    npw = N_CLAUSES // NW
    for b in range(npw // BLK):
        jobs.append((ch, ct, cr, npw, b, sbuf, b * BLK, out_c))
    npw = N_SINGLES // NW
    for b in range(npw // BLK):
        jobs.append((ph, pt, pr, npw, b, sbuf, b * BLK, out_p))
    npw = N_CONJ // NW
    for b in range(npw // BLK):
        jobs.append((c1h, c1t, c1r, npw, b, sbuf, b * BLK, None))
    for b in range(npw // BLK):
        jobs.append((c2h, c2t, c2r, npw, b, sbuf2, b * BLK, None))

    def issue(job, slot):
        h_hbm, t_hbm, r_hbm, npw, b, _, _, _ = job
        base = wid * npw + b * BLK
        hi, ti, ri = idxs[slot]
        hr, tr, rr = rows[slot]
        pltpu.sync_copy(h_hbm.at[pl.ds(base, BLK)], hi)
        pltpu.sync_copy(t_hbm.at[pl.ds(base, BLK)], ti)
        pltpu.sync_copy(r_hbm.at[pl.ds(base, BLK)], ri)
        pltpu.async_copy(ent.at[hi], hr, sem)
        pltpu.async_copy(ent.at[ti], tr, sem)
        pltpu.async_copy(rel.at[ri], rr, sem)

    def drain(slot):
        hr, tr, rr = rows[slot]
        pltpu.make_async_copy(ent.at[pl.ds(0, BLK)], hr, sem).wait()
        pltpu.make_async_copy(ent.at[pl.ds(0, BLK)], tr, sem).wait()
        pltpu.make_async_copy(rel.at[pl.ds(0, BLK)], rr, sem).wait()

    def compute(job, slot):
        hr, tr, rr = rows[slot]
        out_buf, s_off = job[5], job[6]
        for g in range(BLK // LANES):

            def a_body(j, svec):
                a = g * LANES + j
                acc = jnp.zeros((LANES,), jnp.float32)
                for k in range(DIM // LANES):
                    sl = pl.ds(k * LANES, LANES)
                    acc = acc + jnp.abs(hr[a, sl] + rr[a, sl] - tr[a, sl])
                s = GAMMA - jnp.sum(acc)
                return jnp.where(lane == j, s, svec)

            svec = lax.fori_loop(0, LANES, a_body,
                                 jnp.zeros((LANES,), jnp.float32))
            out_buf[pl.ds(s_off + g * LANES, LANES)] = svec

    issue(jobs[0], 0)
    for i, job in enumerate(jobs):
        slot = i % 2
        if i + 1 < len(jobs):
            issue(jobs[i + 1], 1 - slot)
        drain(slot)
        compute(job, slot)
        h_hbm, t_hbm, r_hbm, npw, b, _, s_off, out_hbm = job
        if out_hbm is not None:
            base = wid * npw + b * BLK
            pltpu.sync_copy(sbuf.at[pl.ds(s_off, BLK)],
                            out_hbm.at[pl.ds(base, BLK)])

    npw = N_CONJ // NW
    for v in range(npw // LANES):
        sl = pl.ds(v * LANES, LANES)
        sbuf[sl] = jnp.minimum(sbuf[sl], sbuf2[sl])
    for b in range(npw // BLK):
        base = wid * npw + b * BLK
        pltpu.sync_copy(sbuf.at[pl.ds(b * BLK, BLK)],
                        out_j.at[pl.ds(base, BLK)])


@functools.cache
def _build_transpose():
    mesh = plsc.VectorSubcoreMesh(core_axis_name="c", subcore_axis_name="s")
    return pl.kernel(
        _transpose_kernel,
        mesh=mesh,
        compiler_params=pltpu.CompilerParams(
            needs_layout_passes=False, use_tc_tiling_on_sc=True),
        out_type=jax.ShapeDtypeStruct((N_ENT * DIM,), jnp.float32),
        scratch_types=[
            pltpu.VMEM((DIM, DC_ENT), jnp.float32),
            pltpu.VMEM((DIM, DC_ENT), jnp.float32),
            pltpu.VMEM((DC_ENT * DIM,), jnp.float32),
            pltpu.VMEM((DC_ENT * DIM,), jnp.float32),
            pltpu.VMEM((N_TAIL * DIM,), jnp.float32),
            pltpu.SemaphoreType.DMA,
            pltpu.SemaphoreType.DMA,
        ],
    )


@functools.cache
def _build_scores():
    mesh = plsc.VectorSubcoreMesh(core_axis_name="c", subcore_axis_name="s")
    return pl.kernel(
        _scores_kernel,
        mesh=mesh,
        compiler_params=pltpu.CompilerParams(
            needs_layout_passes=False, use_tc_tiling_on_sc=False),
        out_type=[
            jax.ShapeDtypeStruct((N_CLAUSES,), jnp.float32),
            jax.ShapeDtypeStruct((N_SINGLES,), jnp.float32),
            jax.ShapeDtypeStruct((N_CONJ,), jnp.float32),
        ],
        scratch_types=[
            pltpu.VMEM((BLK,), jnp.int32),
            pltpu.VMEM((BLK,), jnp.int32),
            pltpu.VMEM((BLK,), jnp.int32),
            pltpu.VMEM((BLK, DIM), jnp.float32),
            pltpu.VMEM((BLK, DIM), jnp.float32),
            pltpu.VMEM((BLK, DIM), jnp.float32),
            pltpu.VMEM((BLK,), jnp.int32),
            pltpu.VMEM((BLK,), jnp.int32),
            pltpu.VMEM((BLK,), jnp.int32),
            pltpu.VMEM((BLK, DIM), jnp.float32),
            pltpu.VMEM((BLK, DIM), jnp.float32),
            pltpu.VMEM((BLK, DIM), jnp.float32),
            pltpu.VMEM((N_CLAUSES // NW,), jnp.float32),
            pltpu.VMEM((N_CLAUSES // NW,), jnp.float32),
            pltpu.SemaphoreType.DMA,
        ],
    )


def kernel(clause_entity_embedding, relation_embedding,
           concl_heads, concl_tails, concl_rel,
           premise_heads, premise_tails, premise_rel,
           conj_premise_heads1, conj_premise_tails1, conj_premise_rel1,
           conj_premise_heads2, conj_premise_tails2, conj_premise_rel2):
    ent_t = clause_entity_embedding.T
    tail_flat = lax.slice(clause_entity_embedding,
                          (N_COLS * 128, 0), (N_ENT, DIM)).reshape(-1)
    ent_flat = _build_transpose()(ent_t, tail_flat)
    ent_rm = ent_flat.reshape(N_ENT, DIM)
    concl, prem, conj = _build_scores()(
        ent_rm, relation_embedding,
        concl_heads, concl_tails, concl_rel,
        premise_heads, premise_tails, premise_rel,
        conj_premise_heads1, conj_premise_tails1, conj_premise_rel1,
        conj_premise_heads2, conj_premise_tails2, conj_premise_rel2)
    return (concl, prem, conj)
